# baseline (speedup 1.0000x reference)
import jax
import jax.numpy as jnp
from jax.experimental import pallas as pl
from jax.experimental.pallas import tpu as pltpu

_N = 4096
_CE = 4
_CN = 128
_COUT = 128

_BI = 128
_NI = _N // _BI


def _body(a_ref, wp_ref, x_ref, wself_ref, b_ref, dinv_ref, o_ref):
    x = a_ref[:, 0]
    for t in range(1, 16):
        x = x + a_ref[:, t]
    x = x[:, :4, :] + x[:, 4:, :]
    acc = jnp.sum(x, axis=2)
    msg = (
        jnp.dot(acc, wp_ref[...], preferred_element_type=jnp.float32)
        * dinv_ref[...]
    )
    self_t = jnp.dot(
        x_ref[...], wself_ref[...], preferred_element_type=jnp.float32
    )
    o_ref[...] = jnp.maximum(msg + self_t + b_ref[...], 0.0)


def kernel(D, A, X, W_pass, b_pass, W_self, b_self):
    A4 = (
        A.reshape(_N, 16, 2, 128, _CE)
        .transpose(0, 1, 2, 4, 3)
        .reshape(_N, 16, 8, 128)
    )
    Wp_T = W_pass.T
    Wself_T = W_self.T
    b = (b_pass + b_self).reshape(1, _COUT)
    Dinv = (1.0 / D).reshape(_N, 1)

    out = pl.pallas_call(
        _body,
        grid=(_NI,),
        in_specs=[
            pl.BlockSpec((_BI, 16, 8, 128), lambda i: (i, 0, 0, 0)),
            pl.BlockSpec((_CE, _COUT), lambda i: (0, 0)),
            pl.BlockSpec((_BI, _CN), lambda i: (i, 0)),
            pl.BlockSpec((_CN, _COUT), lambda i: (0, 0)),
            pl.BlockSpec((1, _COUT), lambda i: (0, 0)),
            pl.BlockSpec((_BI, 1), lambda i: (i, 0)),
        ],
        out_specs=pl.BlockSpec((_BI, _COUT), lambda i: (i, 0)),
        out_shape=jax.ShapeDtypeStruct((_N, _COUT), jnp.float32),
        compiler_params=pltpu.CompilerParams(
            dimension_semantics=("parallel",),
        ),
    )(A4, Wp_T, X, Wself_T, b, Dinv)
    return out


# speedup vs baseline: 1.0177x; 1.0177x over previous
import jax
import jax.numpy as jnp
from jax.experimental import pallas as pl
from jax.experimental.pallas import tpu as pltpu

_N = 4096
_CE = 4
_CN = 128
_COUT = 128

_BI = 128
_NI = _N // _BI


def _body(a0_ref, a1_ref, wp_ref, x_ref, wself_ref, b_ref, dinv_ref, o_ref):
    x = a0_ref[:, 0] + a1_ref[:, 0]
    for t in range(1, 8):
        x = x + a0_ref[:, t] + a1_ref[:, t]
    x = x[:, :4, :] + x[:, 4:, :]
    acc = jnp.sum(x, axis=2)
    msg = (
        jnp.dot(acc, wp_ref[...], preferred_element_type=jnp.float32)
        * dinv_ref[...]
    )
    self_t = jnp.dot(
        x_ref[...], wself_ref[...], preferred_element_type=jnp.float32
    )
    o_ref[...] = jnp.maximum(msg + self_t + b_ref[...], 0.0)


def kernel(D, A, X, W_pass, b_pass, W_self, b_self):
    A4 = (
        A.reshape(_N, 16, 2, 128, _CE)
        .transpose(0, 1, 2, 4, 3)
        .reshape(_N, 16, 8, 128)
    )
    Wp_T = W_pass.T
    Wself_T = W_self.T
    b = (b_pass + b_self).reshape(1, _COUT)
    Dinv = (1.0 / D).reshape(_N, 1)

    out = pl.pallas_call(
        _body,
        grid=(_NI,),
        in_specs=[
            pl.BlockSpec((_BI, 8, 8, 128), lambda i: (i, 0, 0, 0)),
            pl.BlockSpec((_BI, 8, 8, 128), lambda i: (i, 1, 0, 0)),
            pl.BlockSpec((_CE, _COUT), lambda i: (0, 0)),
            pl.BlockSpec((_BI, _CN), lambda i: (i, 0)),
            pl.BlockSpec((_CN, _COUT), lambda i: (0, 0)),
            pl.BlockSpec((1, _COUT), lambda i: (0, 0)),
            pl.BlockSpec((_BI, 1), lambda i: (i, 0)),
        ],
        out_specs=pl.BlockSpec((_BI, _COUT), lambda i: (i, 0)),
        out_shape=jax.ShapeDtypeStruct((_N, _COUT), jnp.float32),
        compiler_params=pltpu.CompilerParams(
            dimension_semantics=("parallel",),
        ),
    )(A4, A4, Wp_T, X, Wself_T, b, Dinv)
    return out


# speedup vs baseline: 1.0179x; 1.0002x over previous
import jax
import jax.numpy as jnp
from jax.experimental import pallas as pl
from jax.experimental.pallas import tpu as pltpu

_N = 4096
_CE = 4
_CN = 128
_COUT = 128

_BI = 128
_NI = _N // _BI


def _body(a0_ref, a1_ref, a2_ref, a3_ref, wp_ref, x_ref, wself_ref, b_ref, dinv_ref, o_ref):
    x = (a0_ref[:, 0] + a1_ref[:, 0]) + (a2_ref[:, 0] + a3_ref[:, 0])
    for t in range(1, 4):
        x = x + (a0_ref[:, t] + a1_ref[:, t]) + (a2_ref[:, t] + a3_ref[:, t])
    x = x[:, :4, :] + x[:, 4:, :]
    acc = jnp.sum(x, axis=2)
    msg = (
        jnp.dot(acc, wp_ref[...], preferred_element_type=jnp.float32)
        * dinv_ref[...]
    )
    self_t = jnp.dot(
        x_ref[...], wself_ref[...], preferred_element_type=jnp.float32
    )
    o_ref[...] = jnp.maximum(msg + self_t + b_ref[...], 0.0)


def kernel(D, A, X, W_pass, b_pass, W_self, b_self):
    A4 = (
        A.reshape(_N, 16, 2, 128, _CE)
        .transpose(0, 1, 2, 4, 3)
        .reshape(_N, 16, 8, 128)
    )
    Wp_T = W_pass.T
    Wself_T = W_self.T
    b = (b_pass + b_self).reshape(1, _COUT)
    Dinv = (1.0 / D).reshape(_N, 1)

    out = pl.pallas_call(
        _body,
        grid=(_NI,),
        in_specs=[
            pl.BlockSpec((_BI, 4, 8, 128), lambda i: (i, 0, 0, 0)),
            pl.BlockSpec((_BI, 4, 8, 128), lambda i: (i, 1, 0, 0)),
            pl.BlockSpec((_BI, 4, 8, 128), lambda i: (i, 2, 0, 0)),
            pl.BlockSpec((_BI, 4, 8, 128), lambda i: (i, 3, 0, 0)),
            pl.BlockSpec((_CE, _COUT), lambda i: (0, 0)),
            pl.BlockSpec((_BI, _CN), lambda i: (i, 0)),
            pl.BlockSpec((_CN, _COUT), lambda i: (0, 0)),
            pl.BlockSpec((1, _COUT), lambda i: (0, 0)),
            pl.BlockSpec((_BI, 1), lambda i: (i, 0)),
        ],
        out_specs=pl.BlockSpec((_BI, _COUT), lambda i: (i, 0)),
        out_shape=jax.ShapeDtypeStruct((_N, _COUT), jnp.float32),
        compiler_params=pltpu.CompilerParams(
            dimension_semantics=("parallel",),
        ),
    )(A4, A4, A4, A4, Wp_T, X, Wself_T, b, Dinv)
    return out
